# baseline (speedup 1.0000x reference)
import functools

import jax
import jax.numpy as jnp
from jax import lax
from jax.experimental import pallas as pl
from jax.experimental.pallas import tpu as pltpu
from jax.experimental.pallas import tpu_sc as plsc

_NC = 2
_NS = 16
_NW = _NC * _NS
_B = 16384
_K = 32
_L = 16
_BPW = _B // _NW
_CH = 128
_NCH = _BPW // _CH
_GPC = _CH // _L


def _body(x_hbm, w_hbm, h_hbm, out_hbm, xy, idx_u, idx_v, dst_u, dst_v,
          rows_u, rows_v, sem):
    wid = lax.axis_index("s") * _NC + lax.axis_index("c")
    base = wid * _BPW
    pltpu.sync_copy(x_hbm.at[pl.ds(base * 2, _BPW * 2)], xy)
    lane = lax.iota(jnp.int32, _L)
    for j in range(_NCH):
        for g in range(_GPC):
            p = j * _CH + g * _L
            src = (lane + p) * 2
            idx_u[j, pl.ds(g * _L, _L)] = plsc.load_gather(xy, [src])
            idx_v[j, pl.ds(g * _L, _L)] = plsc.load_gather(xy, [src + 1])
            drow = (base + p + lane) * 2
            dst_u[j, pl.ds(g * _L, _L)] = drow
            dst_v[j, pl.ds(g * _L, _L)] = drow + 1
    copies = []
    for j in range(_NCH):
        s = pl.ds(j * _CH, _CH)
        copies.append(pltpu.async_copy(w_hbm.at[idx_u.at[j]], rows_u.at[s], sem))
        copies.append(pltpu.async_copy(h_hbm.at[idx_v.at[j]], rows_v.at[s], sem))
    for c in copies:
        c.wait()
    copies = []
    for j in range(_NCH):
        s = pl.ds(j * _CH, _CH)
        copies.append(pltpu.async_copy(rows_u.at[s], out_hbm.at[dst_u.at[j]], sem))
        copies.append(pltpu.async_copy(rows_v.at[s], out_hbm.at[dst_v.at[j]], sem))
    for c in copies:
        c.wait()


@functools.partial(
    pl.kernel,
    mesh=plsc.VectorSubcoreMesh(core_axis_name="c", subcore_axis_name="s"),
    compiler_params=pltpu.CompilerParams(
        needs_layout_passes=False, use_tc_tiling_on_sc=False
    ),
    out_type=jax.ShapeDtypeStruct((2 * _B, _K), jnp.float32),
    scratch_types=[
        pltpu.VMEM((2 * _BPW,), jnp.int32),
        pltpu.VMEM((_NCH, _CH), jnp.int32),
        pltpu.VMEM((_NCH, _CH), jnp.int32),
        pltpu.VMEM((_NCH, _CH), jnp.int32),
        pltpu.VMEM((_NCH, _CH), jnp.int32),
        pltpu.VMEM((_BPW, _K), jnp.float32),
        pltpu.VMEM((_BPW, _K), jnp.float32),
        pltpu.SemaphoreType.DMA,
    ],
)
def _lookup(x_hbm, w_hbm, h_hbm, out_hbm, xy, idx_u, idx_v, dst_u, dst_v,
            rows_u, rows_v, sem):
    _body(x_hbm, w_hbm, h_hbm, out_hbm, xy, idx_u, idx_v, dst_u, dst_v,
          rows_u, rows_v, sem)


def kernel(x, W, H):
    out = _lookup(x.reshape(-1), W, H)
    return out.reshape(_B, 2 * _K)
